# Initial kernel scaffold; baseline (speedup 1.0000x reference)
import jax
import jax.numpy as jnp
from jax.experimental import pallas as pl


def kernel(x, edge_attr, en_W1, en_b1, en_W2, en_b2, en_g, en_beta, ee_W1, ee_b1, ee_W2, ee_b2, ee_g, ee_beta, pe_W1, pe_b1, pe_W2, pe_b2, pe_g, pe_beta, pn_W1, pn_b1, pn_W2, pn_b2, pn_g, pn_beta, d_W1, d_b1, d_W2, d_b2, edge_index):
    raise NotImplementedError("write your pallas kernel here")


# speedup vs baseline: 2.3513x; 2.3513x over previous
import functools

import jax
import jax.numpy as jnp
from jax import lax
from jax.experimental import pallas as pl
from jax.experimental.pallas import tpu as pltpu
from jax.experimental.pallas import tpu_sc as plsc

N = 10000
E = 320000
F = 128
S = 5
OUT = 3

NC = 2
NS = 16
NW = NC * NS
EPW = E // NW
CH = 80
NCH = EPW // CH

BE = 512
BN = 1000

_EPS = 1e-5


def _mlp_ln(xin, W1_ref, b1_ref, W2_ref, b2_ref, g_ref, beta_ref):
    h = jnp.maximum(
        jnp.dot(xin, W1_ref[...], preferred_element_type=jnp.float32)
        + b1_ref[...], 0.0)
    y = (jnp.dot(h, W2_ref[...], preferred_element_type=jnp.float32)
         + b2_ref[...])
    mu = jnp.mean(y, axis=-1, keepdims=True)
    d = y - mu
    var = jnp.mean(d * d, axis=-1, keepdims=True)
    return d * lax.rsqrt(var + _EPS) * g_ref[...] + beta_ref[...]


def _enc_body(x_ref, W1, b1, W2, b2, g, beta, o_ref):
    o_ref[...] = _mlp_ln(x_ref[...], W1, b1, W2, b2, g, beta)


def _edge_body(el_ref, g1_ref, g2_ref, W1, b1, W2, b2, g, beta,
               enew_ref, elnew_ref):
    el = el_ref[...]
    e_in = el + g1_ref[...] + g2_ref[...]
    e_new = _mlp_ln(e_in, W1, b1, W2, b2, g, beta)
    enew_ref[...] = e_new
    elnew_ref[...] = el + e_new


def _edge_body_last(el_ref, g1_ref, g2_ref, W1, b1, W2, b2, g, beta,
                    enew_ref):
    e_in = el_ref[...] + g1_ref[...] + g2_ref[...]
    enew_ref[...] = _mlp_ln(e_in, W1, b1, W2, b2, g, beta)


def _node_body(xl_ref, p0_ref, p1_ref, W1, b1, W2, b2, g, beta, o_ref):
    xl = xl_ref[...]
    t = xl + p0_ref[...] + p1_ref[...]
    o_ref[...] = xl + _mlp_ln(t, W1, b1, W2, b2, g, beta)


def _dec_body(xl_ref, W1, b1, W2, b2, o_ref):
    h = jnp.maximum(
        jnp.dot(xl_ref[...], W1[...], preferred_element_type=jnp.float32)
        + b1[...], 0.0)
    o_ref[...] = (jnp.dot(h, W2[...], preferred_element_type=jnp.float32)
                  + b2[...])


def _row_spec(B):
    return pl.BlockSpec((B, F), lambda i: (i, 0))


def _w_spec():
    return pl.BlockSpec((F, F), lambda i: (0, 0))


def _v_spec():
    return pl.BlockSpec((1, F), lambda i: (0, 0))


def _tc_encode(x, W1, b1, W2, b2, g, beta, B):
    R = x.shape[0]
    return pl.pallas_call(
        _enc_body,
        grid=(R // B,),
        in_specs=[_row_spec(B), _w_spec(), _v_spec(), _w_spec(), _v_spec(),
                  _v_spec(), _v_spec()],
        out_specs=_row_spec(B),
        out_shape=jax.ShapeDtypeStruct((R, F), jnp.float32),
    )(x, W1, b1.reshape(1, F), W2, b2.reshape(1, F),
      g.reshape(1, F), beta.reshape(1, F))


def _tc_edge(el, g1, g2, W1, b1, W2, b2, g, beta, last):
    body = _edge_body_last if last else _edge_body
    n_out = 1 if last else 2
    shp = jax.ShapeDtypeStruct((E, F), jnp.float32)
    out = pl.pallas_call(
        body,
        grid=(E // BE,),
        in_specs=[_row_spec(BE)] * 3 + [_w_spec(), _v_spec(), _w_spec(),
                                        _v_spec(), _v_spec(), _v_spec()],
        out_specs=[_row_spec(BE)] * n_out,
        out_shape=[shp] * n_out,
    )(el, g1, g2, W1, b1.reshape(1, F), W2, b2.reshape(1, F),
      g.reshape(1, F), beta.reshape(1, F))
    if last:
        return out[0], None
    return out[0], out[1]


def _tc_node(xl, p0, p1, W1, b1, W2, b2, g, beta):
    return pl.pallas_call(
        _node_body,
        grid=(N // BN,),
        in_specs=[_row_spec(BN)] * 3 + [_w_spec(), _v_spec(), _w_spec(),
                                        _v_spec(), _v_spec(), _v_spec()],
        out_specs=_row_spec(BN),
        out_shape=jax.ShapeDtypeStruct((N, F), jnp.float32),
    )(xl, p0, p1, W1, b1.reshape(1, F), W2, b2.reshape(1, F),
      g.reshape(1, F), beta.reshape(1, F))


def _tc_decode(xl, W1, b1, W2p, b2p):
    return pl.pallas_call(
        _dec_body,
        grid=(N // BN,),
        in_specs=[_row_spec(BN), _w_spec(), _v_spec(), _w_spec(), _v_spec()],
        out_specs=_row_spec(BN),
        out_shape=jax.ShapeDtypeStruct((N, F), jnp.float32),
    )(xl, W1, b1.reshape(1, F), W2p, b2p.reshape(1, F))


@functools.cache
def _sc_kernels():
    mesh = plsc.VectorSubcoreMesh(core_axis_name="c", subcore_axis_name="s",
                                  num_cores=NC, num_subcores=NS)

    @functools.partial(
        pl.kernel,
        out_type=(jax.ShapeDtypeStruct((E, F), jnp.float32),
                  jax.ShapeDtypeStruct((E, F), jnp.float32)),
        mesh=mesh,
        scratch_types=[
            pltpu.VMEM((NCH, CH), jnp.int32),
            pltpu.VMEM((NCH, CH), jnp.int32),
            pltpu.VMEM((CH, F), jnp.float32),
            pltpu.VMEM((CH, F), jnp.float32),
            pltpu.SemaphoreType.DMA,
            pltpu.SemaphoreType.DMA,
        ],
    )
    def sc_gather(xl_hbm, src_hbm, dst_hbm, g1_hbm, g2_hbm,
                  idx_s, idx_d, buf_s, buf_d, sem_s, sem_d):
        cid = lax.axis_index("c")
        sid = lax.axis_index("s")
        wid = sid * NC + cid
        base = wid * EPW
        pltpu.sync_copy(src_hbm.at[wid], idx_s)
        pltpu.sync_copy(dst_hbm.at[wid], idx_d)

        def body(j, carry):
            cp_s = pltpu.async_copy(xl_hbm.at[idx_s.at[j]], buf_s, sem_s)
            cp_d = pltpu.async_copy(xl_hbm.at[idx_d.at[j]], buf_d, sem_d)
            cp_s.wait()
            cp_d.wait()
            row0 = base + j * CH
            pltpu.sync_copy(buf_s, g1_hbm.at[pl.ds(row0, CH)])
            pltpu.sync_copy(buf_d, g2_hbm.at[pl.ds(row0, CH)])
            return carry

        lax.fori_loop(0, NCH, body, 0)

    @functools.partial(
        pl.kernel,
        out_type=jax.ShapeDtypeStruct((NC, N, F), jnp.float32),
        mesh=mesh,
        scratch_types=[
            pltpu.VMEM((NCH, CH), jnp.int32),
            pltpu.VMEM((CH, F), jnp.float32),
            pltpu.VMEM_SHARED((N, F), jnp.float32),
            pltpu.SemaphoreType.DMA,
        ],
    )
    def sc_scatter(enew_hbm, dst_hbm, zeros_hbm, out_hbm,
                   idx_d, buf, acc, sem):
        cid = lax.axis_index("c")
        sid = lax.axis_index("s")
        wid = sid * NC + cid
        base = wid * EPW
        rps = N // 10
        @pl.when(sid < 10)
        def _zero():
            pltpu.sync_copy(zeros_hbm, acc.at[pl.ds(sid * rps, rps)])
        pltpu.sync_copy(dst_hbm.at[wid], idx_d)
        plsc.subcore_barrier()

        def body(j, carry):
            pltpu.async_copy(enew_hbm.at[pl.ds(base + j * CH, CH)], buf,
                             sem).wait()
            pltpu.sync_copy(buf, acc.at[idx_d.at[j]], add=True)
            return carry

        lax.fori_loop(0, NCH, body, 0)
        plsc.subcore_barrier()

        @pl.when(sid < 10)
        def _writeback():
            pltpu.sync_copy(acc.at[pl.ds(sid * rps, rps)],
                            out_hbm.at[cid, pl.ds(sid * rps, rps)])

    return sc_gather, sc_scatter


def kernel(x, edge_attr, en_W1, en_b1, en_W2, en_b2, en_g, en_beta,
           ee_W1, ee_b1, ee_W2, ee_b2, ee_g, ee_beta,
           pe_W1, pe_b1, pe_W2, pe_b2, pe_g, pe_beta,
           pn_W1, pn_b1, pn_W2, pn_b2, pn_g, pn_beta,
           d_W1, d_b1, d_W2, d_b2, edge_index):
    src3 = edge_index[0].reshape(NW, NCH, CH)
    dst3 = edge_index[1].reshape(NW, NCH, CH)
    zeros = jnp.zeros((N // 10, F), jnp.float32)

    xl = _tc_encode(x, en_W1, en_b1, en_W2, en_b2, en_g, en_beta, BN)
    el = _tc_encode(edge_attr, ee_W1, ee_b1, ee_W2, ee_b2, ee_g, ee_beta, BE)

    sc_gather, sc_scatter = _sc_kernels()
    for s in range(S):
        g1, g2 = sc_gather(xl, src3, dst3)
        e_new, el = _tc_edge(el, g1, g2, pe_W1[s], pe_b1[s], pe_W2[s],
                             pe_b2[s], pe_g[s], pe_beta[s], last=(s == S - 1))
        p = sc_scatter(e_new, dst3, zeros)
        xl = _tc_node(xl, p[0], p[1], pn_W1[s], pn_b1[s], pn_W2[s],
                      pn_b2[s], pn_g[s], pn_beta[s])

    W2p = jnp.pad(d_W2, ((0, 0), (0, F - OUT)))
    b2p = jnp.pad(d_b2, (0, F - OUT))
    out = _tc_decode(xl, d_W1, d_b1, W2p, b2p)
    return out[:, :OUT]


# speedup vs baseline: 2.6376x; 1.1217x over previous
import functools

import jax
import jax.numpy as jnp
from jax import lax
from jax.experimental import pallas as pl
from jax.experimental.pallas import tpu as pltpu
from jax.experimental.pallas import tpu_sc as plsc

N = 10000
E = 320000
F = 128
S = 5
OUT = 3

NC = 2
NS = 16
NW = NC * NS
EPW = E // NW
CH = 80
NCH = EPW // CH
GRP = 5
ROWS = GRP * CH
NOUT = NCH // GRP

BE = 512
BN = 1000

_EPS = 1e-5


def _mlp_ln(xin, W1_ref, b1_ref, W2_ref, b2_ref, g_ref, beta_ref):
    h = jnp.maximum(
        jnp.dot(xin, W1_ref[...], preferred_element_type=jnp.float32)
        + b1_ref[...], 0.0)
    y = (jnp.dot(h, W2_ref[...], preferred_element_type=jnp.float32)
         + b2_ref[...])
    mu = jnp.mean(y, axis=-1, keepdims=True)
    d = y - mu
    var = jnp.mean(d * d, axis=-1, keepdims=True)
    return d * lax.rsqrt(var + _EPS) * g_ref[...] + beta_ref[...]


def _enc_body(x_ref, W1, b1, W2, b2, g, beta, o_ref):
    o_ref[...] = _mlp_ln(x_ref[...], W1, b1, W2, b2, g, beta)


def _edge_body(el_ref, g1_ref, g2_ref, W1, b1, W2, b2, g, beta,
               enew_ref, elnew_ref):
    el = el_ref[...]
    e_in = el + g1_ref[...] + g2_ref[...]
    e_new = _mlp_ln(e_in, W1, b1, W2, b2, g, beta)
    enew_ref[...] = e_new
    elnew_ref[...] = el + e_new


def _edge_body_last(el_ref, g1_ref, g2_ref, W1, b1, W2, b2, g, beta,
                    enew_ref):
    e_in = el_ref[...] + g1_ref[...] + g2_ref[...]
    enew_ref[...] = _mlp_ln(e_in, W1, b1, W2, b2, g, beta)


def _node_body(xl_ref, p0_ref, p1_ref, W1, b1, W2, b2, g, beta, o_ref):
    xl = xl_ref[...]
    t = xl + p0_ref[...] + p1_ref[...]
    o_ref[...] = xl + _mlp_ln(t, W1, b1, W2, b2, g, beta)


def _dec_body(xl_ref, W1, b1, W2, b2, o_ref):
    h = jnp.maximum(
        jnp.dot(xl_ref[...], W1[...], preferred_element_type=jnp.float32)
        + b1[...], 0.0)
    o_ref[...] = (jnp.dot(h, W2[...], preferred_element_type=jnp.float32)
                  + b2[...])


def _row_spec(B):
    return pl.BlockSpec((B, F), lambda i: (i, 0))


def _w_spec():
    return pl.BlockSpec((F, F), lambda i: (0, 0))


def _v_spec():
    return pl.BlockSpec((1, F), lambda i: (0, 0))


def _tc_encode(x, W1, b1, W2, b2, g, beta, B):
    R = x.shape[0]
    return pl.pallas_call(
        _enc_body,
        grid=(R // B,),
        in_specs=[_row_spec(B), _w_spec(), _v_spec(), _w_spec(), _v_spec(),
                  _v_spec(), _v_spec()],
        out_specs=_row_spec(B),
        out_shape=jax.ShapeDtypeStruct((R, F), jnp.float32),
    )(x, W1, b1.reshape(1, F), W2, b2.reshape(1, F),
      g.reshape(1, F), beta.reshape(1, F))


def _tc_edge(el, g1, g2, W1, b1, W2, b2, g, beta, last):
    body = _edge_body_last if last else _edge_body
    n_out = 1 if last else 2
    shp = jax.ShapeDtypeStruct((E, F), jnp.float32)
    out = pl.pallas_call(
        body,
        grid=(E // BE,),
        in_specs=[_row_spec(BE)] * 3 + [_w_spec(), _v_spec(), _w_spec(),
                                        _v_spec(), _v_spec(), _v_spec()],
        out_specs=[_row_spec(BE)] * n_out,
        out_shape=[shp] * n_out,
    )(el, g1, g2, W1, b1.reshape(1, F), W2, b2.reshape(1, F),
      g.reshape(1, F), beta.reshape(1, F))
    if last:
        return out[0], None
    return out[0], out[1]


def _tc_node(xl, p0, p1, W1, b1, W2, b2, g, beta):
    return pl.pallas_call(
        _node_body,
        grid=(N // BN,),
        in_specs=[_row_spec(BN)] * 3 + [_w_spec(), _v_spec(), _w_spec(),
                                        _v_spec(), _v_spec(), _v_spec()],
        out_specs=_row_spec(BN),
        out_shape=jax.ShapeDtypeStruct((N, F), jnp.float32),
    )(xl, p0, p1, W1, b1.reshape(1, F), W2, b2.reshape(1, F),
      g.reshape(1, F), beta.reshape(1, F))


def _tc_decode(xl, W1, b1, W2p, b2p):
    return pl.pallas_call(
        _dec_body,
        grid=(N // BN,),
        in_specs=[_row_spec(BN), _w_spec(), _v_spec(), _w_spec(), _v_spec()],
        out_specs=_row_spec(BN),
        out_shape=jax.ShapeDtypeStruct((N, F), jnp.float32),
    )(xl, W1, b1.reshape(1, F), W2p, b2p.reshape(1, F))


@functools.cache
def _sc_kernels():
    mesh = plsc.VectorSubcoreMesh(core_axis_name="c", subcore_axis_name="s",
                                  num_cores=NC, num_subcores=NS)

    @functools.partial(
        pl.kernel,
        out_type=jax.ShapeDtypeStruct((E, F), jnp.float32),
        mesh=mesh,
        scratch_types=[
            pltpu.VMEM((NCH, CH), jnp.int32),
            pltpu.VMEM((2, ROWS, F), jnp.float32),
            pltpu.SemaphoreType.DMA,
            pltpu.SemaphoreType.DMA,
            pltpu.SemaphoreType.DMA,
            pltpu.SemaphoreType.DMA,
        ],
    )
    def sc_gather(xl_hbm, ei_hbm, g_hbm,
                  idx, buf, semg0, semg1, semw0, semw1):
        cid = lax.axis_index("c")
        sid = lax.axis_index("s")
        wid = sid * NC + cid
        base = wid * EPW
        pltpu.sync_copy(ei_hbm.at[wid], idx)
        semg = (semg0, semg1)
        semw = (semw0, semw1)

        def issue(grp, slot, sem):
            for b in range(GRP):
                pltpu.async_copy(xl_hbm.at[idx.at[grp * GRP + b]],
                                 buf.at[slot, pl.ds(b * CH, CH)], sem)

        def drain_gathers(slot, sem):
            for b in range(GRP):
                pltpu.make_async_copy(
                    xl_hbm.at[idx.at[0]],
                    buf.at[slot, pl.ds(b * CH, CH)], sem).wait()

        def drain_write(slot, sem):
            pltpu.make_async_copy(buf.at[slot],
                                  g_hbm.at[pl.ds(0, ROWS)], sem).wait()

        issue(0, 0, semg[0])

        @pl.loop(0, NOUT, step=2)
        def outer(j):
            for b in (0, 1):
                cur = j + b
                o = 1 - b

                @pl.when(cur < NOUT)
                def _():
                    @pl.when(cur >= 1)
                    def _():
                        drain_write(o, semw[o])

                    @pl.when(cur + 1 < NOUT)
                    def _():
                        issue(cur + 1, o, semg[o])

                    drain_gathers(b, semg[b])
                    pltpu.async_copy(buf.at[b],
                                     g_hbm.at[pl.ds(base + cur * ROWS, ROWS)],
                                     semw[b])

        drain_write((NOUT - 1) % 2, semw[(NOUT - 1) % 2])

    @functools.partial(
        pl.kernel,
        out_type=jax.ShapeDtypeStruct((NC, N, F), jnp.float32),
        mesh=mesh,
        scratch_types=[
            pltpu.VMEM((NCH, CH), jnp.int32),
            pltpu.VMEM((2, CH, F), jnp.float32),
            pltpu.VMEM_SHARED((N, F), jnp.float32),
            pltpu.SemaphoreType.DMA,
            pltpu.SemaphoreType.DMA,
        ],
    )
    def sc_scatter(enew_hbm, dst_hbm, zeros_hbm, out_hbm,
                   idx_d, buf, acc, semr0, semr1):
        cid = lax.axis_index("c")
        sid = lax.axis_index("s")
        wid = sid * NC + cid
        base = wid * EPW
        rps = N // 10
        @pl.when(sid < 10)
        def _zero():
            pltpu.sync_copy(zeros_hbm, acc.at[pl.ds(sid * rps, rps)])
        pltpu.sync_copy(dst_hbm.at[wid], idx_d)
        plsc.subcore_barrier()
        semr = (semr0, semr1)

        def issue(grp, slot, sem):
            pltpu.async_copy(enew_hbm.at[pl.ds(base + grp * CH, CH)],
                             buf.at[slot], sem)

        def drain(slot, sem):
            pltpu.make_async_copy(enew_hbm.at[pl.ds(0, CH)],
                                  buf.at[slot], sem).wait()

        issue(0, 0, semr[0])

        @pl.loop(0, NCH, step=2)
        def outer(j):
            for b in (0, 1):
                cur = j + b
                o = 1 - b

                @pl.when(cur < NCH)
                def _():
                    @pl.when(cur + 1 < NCH)
                    def _():
                        issue(cur + 1, o, semr[o])

                    drain(b, semr[b])
                    pltpu.sync_copy(buf.at[b], acc.at[idx_d.at[cur]],
                                    add=True)

        plsc.subcore_barrier()

        @pl.when(sid < 10)
        def _writeback():
            pltpu.sync_copy(acc.at[pl.ds(sid * rps, rps)],
                            out_hbm.at[cid, pl.ds(sid * rps, rps)])

    return sc_gather, sc_scatter


def kernel(x, edge_attr, en_W1, en_b1, en_W2, en_b2, en_g, en_beta,
           ee_W1, ee_b1, ee_W2, ee_b2, ee_g, ee_beta,
           pe_W1, pe_b1, pe_W2, pe_b2, pe_g, pe_beta,
           pn_W1, pn_b1, pn_W2, pn_b2, pn_g, pn_beta,
           d_W1, d_b1, d_W2, d_b2, edge_index):
    src3 = edge_index[0].reshape(NW, NCH, CH)
    dst3 = edge_index[1].reshape(NW, NCH, CH)
    zeros = jnp.zeros((N // 10, F), jnp.float32)

    xl = _tc_encode(x, en_W1, en_b1, en_W2, en_b2, en_g, en_beta, BN)
    el = _tc_encode(edge_attr, ee_W1, ee_b1, ee_W2, ee_b2, ee_g, ee_beta, BE)

    sc_gather, sc_scatter = _sc_kernels()
    for s in range(S):
        g1 = sc_gather(xl, src3)
        g2 = sc_gather(xl, dst3)
        e_new, el = _tc_edge(el, g1, g2, pe_W1[s], pe_b1[s], pe_W2[s],
                             pe_b2[s], pe_g[s], pe_beta[s], last=(s == S - 1))
        p = sc_scatter(e_new, dst3, zeros)
        xl = _tc_node(xl, p[0], p[1], pn_W1[s], pn_b1[s], pn_W2[s],
                      pn_b2[s], pn_g[s], pn_beta[s])

    W2p = jnp.pad(d_W2, ((0, 0), (0, F - OUT)))
    b2p = jnp.pad(d_b2, (0, F - OUT))
    out = _tc_decode(xl, d_W1, d_b1, W2p, b2p)
    return out[:, :OUT]


# speedup vs baseline: 2.6390x; 1.0006x over previous
import functools

import jax
import jax.numpy as jnp
from jax import lax
from jax.experimental import pallas as pl
from jax.experimental.pallas import tpu as pltpu
from jax.experimental.pallas import tpu_sc as plsc

N = 10000
E = 320000
F = 128
S = 5
OUT = 3

NC = 2
NS = 16
NW = NC * NS
EPW = E // NW
CH = 80
NCH = EPW // CH
GRP = 5
ROWS = GRP * CH
NOUT = NCH // GRP

BE = 512
BN = 1000

_EPS = 1e-5


def _mlp_ln(xin, W1_ref, b1_ref, W2_ref, b2_ref, g_ref, beta_ref):
    h = jnp.maximum(
        jnp.dot(xin, W1_ref[...], preferred_element_type=jnp.float32)
        + b1_ref[...], 0.0)
    y = (jnp.dot(h, W2_ref[...], preferred_element_type=jnp.float32)
         + b2_ref[...])
    mu = jnp.mean(y, axis=-1, keepdims=True)
    d = y - mu
    var = jnp.mean(d * d, axis=-1, keepdims=True)
    return d * lax.rsqrt(var + _EPS) * g_ref[...] + beta_ref[...]


def _enc_body(x_ref, W1, b1, W2, b2, g, beta, o_ref):
    o_ref[...] = _mlp_ln(x_ref[...], W1, b1, W2, b2, g, beta)


def _edge_body(el_ref, g1_ref, g2_ref, W1, b1, W2, b2, g, beta,
               enew_ref, elnew_ref):
    el = el_ref[...]
    e_in = el + g1_ref[...] + g2_ref[...]
    e_new = _mlp_ln(e_in, W1, b1, W2, b2, g, beta)
    enew_ref[...] = e_new
    elnew_ref[...] = el + e_new


def _edge_body_last(el_ref, g1_ref, g2_ref, W1, b1, W2, b2, g, beta,
                    enew_ref):
    e_in = el_ref[...] + g1_ref[...] + g2_ref[...]
    enew_ref[...] = _mlp_ln(e_in, W1, b1, W2, b2, g, beta)


def _node_body(xl_ref, p0_ref, p1_ref, W1, b1, W2, b2, g, beta, o_ref):
    xl = xl_ref[...]
    t = xl + p0_ref[...] + p1_ref[...]
    o_ref[...] = xl + _mlp_ln(t, W1, b1, W2, b2, g, beta)


def _dec_body(xl_ref, W1, b1, W2, b2, o_ref):
    h = jnp.maximum(
        jnp.dot(xl_ref[...], W1[...], preferred_element_type=jnp.float32)
        + b1[...], 0.0)
    o_ref[...] = (jnp.dot(h, W2[...], preferred_element_type=jnp.float32)
                  + b2[...])


def _row_spec(B):
    return pl.BlockSpec((B, F), lambda i: (i, 0))


def _w_spec():
    return pl.BlockSpec((F, F), lambda i: (0, 0))


def _v_spec():
    return pl.BlockSpec((1, F), lambda i: (0, 0))


def _tc_encode(x, W1, b1, W2, b2, g, beta, B):
    R = x.shape[0]
    return pl.pallas_call(
        _enc_body,
        grid=(R // B,),
        in_specs=[_row_spec(B), _w_spec(), _v_spec(), _w_spec(), _v_spec(),
                  _v_spec(), _v_spec()],
        out_specs=_row_spec(B),
        out_shape=jax.ShapeDtypeStruct((R, F), jnp.float32),
    )(x, W1, b1.reshape(1, F), W2, b2.reshape(1, F),
      g.reshape(1, F), beta.reshape(1, F))


def _tc_edge(el, g1, g2, W1, b1, W2, b2, g, beta, last):
    body = _edge_body_last if last else _edge_body
    n_out = 1 if last else 2
    shp = jax.ShapeDtypeStruct((E, F), jnp.float32)
    out = pl.pallas_call(
        body,
        grid=(E // BE,),
        in_specs=[_row_spec(BE)] * 3 + [_w_spec(), _v_spec(), _w_spec(),
                                        _v_spec(), _v_spec(), _v_spec()],
        out_specs=[_row_spec(BE)] * n_out,
        out_shape=[shp] * n_out,
    )(el, g1, g2, W1, b1.reshape(1, F), W2, b2.reshape(1, F),
      g.reshape(1, F), beta.reshape(1, F))
    if last:
        return out[0], None
    return out[0], out[1]


def _tc_node(xl, p0, p1, W1, b1, W2, b2, g, beta):
    return pl.pallas_call(
        _node_body,
        grid=(N // BN,),
        in_specs=[_row_spec(BN)] * 3 + [_w_spec(), _v_spec(), _w_spec(),
                                        _v_spec(), _v_spec(), _v_spec()],
        out_specs=_row_spec(BN),
        out_shape=jax.ShapeDtypeStruct((N, F), jnp.float32),
    )(xl, p0, p1, W1, b1.reshape(1, F), W2, b2.reshape(1, F),
      g.reshape(1, F), beta.reshape(1, F))


def _tc_decode(xl, W1, b1, W2p, b2p):
    return pl.pallas_call(
        _dec_body,
        grid=(N // BN,),
        in_specs=[_row_spec(BN), _w_spec(), _v_spec(), _w_spec(), _v_spec()],
        out_specs=_row_spec(BN),
        out_shape=jax.ShapeDtypeStruct((N, F), jnp.float32),
    )(xl, W1, b1.reshape(1, F), W2p, b2p.reshape(1, F))


@functools.cache
def _sc_kernels():
    mesh = plsc.VectorSubcoreMesh(core_axis_name="c", subcore_axis_name="s",
                                  num_cores=NC, num_subcores=NS)

    @functools.partial(
        pl.kernel,
        out_type=jax.ShapeDtypeStruct((E, F), jnp.float32),
        mesh=mesh,
        scratch_types=[
            pltpu.VMEM((NCH, CH), jnp.int32),
            pltpu.VMEM((2, ROWS, F), jnp.float32),
            pltpu.SemaphoreType.DMA,
            pltpu.SemaphoreType.DMA,
            pltpu.SemaphoreType.DMA,
            pltpu.SemaphoreType.DMA,
        ],
    )
    def sc_gather(xl_hbm, ei_hbm, g_hbm,
                  idx, buf, semg0, semg1, semw0, semw1):
        cid = lax.axis_index("c")
        sid = lax.axis_index("s")
        wid = sid * NC + cid
        base = wid * EPW
        pltpu.sync_copy(ei_hbm.at[wid], idx)
        semg = (semg0, semg1)
        semw = (semw0, semw1)

        def issue(grp, slot, sem):
            for b in range(GRP):
                pltpu.async_copy(xl_hbm.at[idx.at[grp * GRP + b]],
                                 buf.at[slot, pl.ds(b * CH, CH)], sem)

        def drain_gathers(slot, sem):
            for b in range(GRP):
                pltpu.make_async_copy(
                    xl_hbm.at[idx.at[0]],
                    buf.at[slot, pl.ds(b * CH, CH)], sem).wait()

        def drain_write(slot, sem):
            pltpu.make_async_copy(buf.at[slot],
                                  g_hbm.at[pl.ds(0, ROWS)], sem).wait()

        issue(0, 0, semg[0])

        @pl.loop(0, NOUT, step=2)
        def outer(j):
            for b in (0, 1):
                cur = j + b
                o = 1 - b

                @pl.when(cur < NOUT)
                def _():
                    @pl.when(cur >= 1)
                    def _():
                        drain_write(o, semw[o])

                    @pl.when(cur + 1 < NOUT)
                    def _():
                        issue(cur + 1, o, semg[o])

                    drain_gathers(b, semg[b])
                    pltpu.async_copy(buf.at[b],
                                     g_hbm.at[pl.ds(base + cur * ROWS, ROWS)],
                                     semw[b])

        drain_write((NOUT - 1) % 2, semw[(NOUT - 1) % 2])

    @functools.partial(
        pl.kernel,
        out_type=jax.ShapeDtypeStruct((NC, N, F), jnp.float32),
        mesh=mesh,
        scratch_types=[
            pltpu.VMEM((NCH, CH), jnp.int32),
            pltpu.VMEM((2, CH, F), jnp.float32),
            pltpu.VMEM_SHARED((N, F), jnp.float32),
            pltpu.SemaphoreType.DMA,
            pltpu.SemaphoreType.DMA,
            pltpu.SemaphoreType.DMA,
            pltpu.SemaphoreType.DMA,
        ],
    )
    def sc_scatter(enew_hbm, dst_hbm, zeros_hbm, out_hbm,
                   idx_d, buf, acc, semr0, semr1, sema0, sema1):
        cid = lax.axis_index("c")
        sid = lax.axis_index("s")
        wid = sid * NC + cid
        base = wid * EPW
        rps = N // 10
        @pl.when(sid < 10)
        def _zero():
            pltpu.sync_copy(zeros_hbm, acc.at[pl.ds(sid * rps, rps)])
        pltpu.sync_copy(dst_hbm.at[wid], idx_d)
        plsc.subcore_barrier()
        semr = (semr0, semr1)
        sema = (sema0, sema1)

        def issue(grp, slot):
            pltpu.async_copy(enew_hbm.at[pl.ds(base + grp * CH, CH)],
                             buf.at[slot], semr[slot])

        def drain_read(slot):
            pltpu.make_async_copy(enew_hbm.at[pl.ds(0, CH)],
                                  buf.at[slot], semr[slot]).wait()

        def drain_add(slot):
            pltpu.make_async_copy(buf.at[slot], acc.at[idx_d.at[0]],
                                  sema[slot]).wait()

        issue(0, 0)

        @pl.loop(0, NCH, step=2)
        def outer(j):
            for b in (0, 1):
                cur = j + b
                o = 1 - b

                @pl.when(cur < NCH)
                def _():
                    @pl.when(cur >= 1)
                    def _():
                        drain_add(o)

                    @pl.when(cur + 1 < NCH)
                    def _():
                        issue(cur + 1, o)

                    drain_read(b)
                    pltpu.async_copy(buf.at[b], acc.at[idx_d.at[cur]],
                                     sema[b], add=True)

        drain_add((NCH - 1) % 2)
        plsc.subcore_barrier()

        @pl.when(sid < 10)
        def _writeback():
            pltpu.sync_copy(acc.at[pl.ds(sid * rps, rps)],
                            out_hbm.at[cid, pl.ds(sid * rps, rps)])

    return sc_gather, sc_scatter


def kernel(x, edge_attr, en_W1, en_b1, en_W2, en_b2, en_g, en_beta,
           ee_W1, ee_b1, ee_W2, ee_b2, ee_g, ee_beta,
           pe_W1, pe_b1, pe_W2, pe_b2, pe_g, pe_beta,
           pn_W1, pn_b1, pn_W2, pn_b2, pn_g, pn_beta,
           d_W1, d_b1, d_W2, d_b2, edge_index):
    src3 = edge_index[0].reshape(NW, NCH, CH)
    dst3 = edge_index[1].reshape(NW, NCH, CH)
    zeros = jnp.zeros((N // 10, F), jnp.float32)

    xl = _tc_encode(x, en_W1, en_b1, en_W2, en_b2, en_g, en_beta, BN)
    el = _tc_encode(edge_attr, ee_W1, ee_b1, ee_W2, ee_b2, ee_g, ee_beta, BE)

    sc_gather, sc_scatter = _sc_kernels()
    for s in range(S):
        g1 = sc_gather(xl, src3)
        g2 = sc_gather(xl, dst3)
        e_new, el = _tc_edge(el, g1, g2, pe_W1[s], pe_b1[s], pe_W2[s],
                             pe_b2[s], pe_g[s], pe_beta[s], last=(s == S - 1))
        p = sc_scatter(e_new, dst3, zeros)
        xl = _tc_node(xl, p[0], p[1], pn_W1[s], pn_b1[s], pn_W2[s],
                      pn_b2[s], pn_g[s], pn_beta[s])

    W2p = jnp.pad(d_W2, ((0, 0), (0, F - OUT)))
    b2p = jnp.pad(d_b2, (0, F - OUT))
    out = _tc_decode(xl, d_W1, d_b1, W2p, b2p)
    return out[:, :OUT]


# speedup vs baseline: 3.0540x; 1.1573x over previous
import functools

import jax
import jax.numpy as jnp
from jax import lax
from jax.experimental import pallas as pl
from jax.experimental.pallas import tpu as pltpu
from jax.experimental.pallas import tpu_sc as plsc

N = 10000
E = 320000
F = 128
S = 5
OUT = 3

NC = 2
NS = 16
NW = NC * NS

EH = E // 2
EPW = EH // NW
CH = 40
NCH = EPW // CH
GRP = 5
ROWS = GRP * CH
NOUT = NCH // GRP

BE = 640
BN = 1000

_EPS = 1e-5


def _mlp_ln(xin, W1_ref, b1_ref, W2_ref, b2_ref, g_ref, beta_ref):
    h = jnp.maximum(
        jnp.dot(xin, W1_ref[...], preferred_element_type=jnp.float32)
        + b1_ref[...], 0.0)
    y = (jnp.dot(h, W2_ref[...], preferred_element_type=jnp.float32)
         + b2_ref[...])
    mu = jnp.mean(y, axis=-1, keepdims=True)
    d = y - mu
    var = jnp.mean(d * d, axis=-1, keepdims=True)
    return d * lax.rsqrt(var + _EPS) * g_ref[...] + beta_ref[...]


def _enc_body(x_ref, W1, b1, W2, b2, g, beta, o_ref):
    o_ref[...] = _mlp_ln(x_ref[...], W1, b1, W2, b2, g, beta)


def _edge_body(el_ref, g1_ref, g2_ref, W1, b1, W2, b2, g, beta,
               enew_ref, elnew_ref):
    el = el_ref[...]
    e_in = el + g1_ref[...] + g2_ref[...]
    e_new = _mlp_ln(e_in, W1, b1, W2, b2, g, beta)
    enew_ref[...] = e_new
    elnew_ref[...] = el + e_new


def _edge_body_last(el_ref, g1_ref, g2_ref, W1, b1, W2, b2, g, beta,
                    enew_ref):
    e_in = el_ref[...] + g1_ref[...] + g2_ref[...]
    enew_ref[...] = _mlp_ln(e_in, W1, b1, W2, b2, g, beta)


def _node_body(xl_ref, p0_ref, p1_ref, p2_ref, p3_ref,
               W1, b1, W2, b2, g, beta, o_ref):
    xl = xl_ref[...]
    t = xl + ((p0_ref[...] + p1_ref[...]) + (p2_ref[...] + p3_ref[...]))
    o_ref[...] = xl + _mlp_ln(t, W1, b1, W2, b2, g, beta)


def _dec_body(xl_ref, W1, b1, W2, b2, o_ref):
    h = jnp.maximum(
        jnp.dot(xl_ref[...], W1[...], preferred_element_type=jnp.float32)
        + b1[...], 0.0)
    o_ref[...] = (jnp.dot(h, W2[...], preferred_element_type=jnp.float32)
                  + b2[...])


def _row_spec(B, off=0):
    return pl.BlockSpec((B, F), lambda i, _o=off: (i + _o, 0))


def _w_spec():
    return pl.BlockSpec((F, F), lambda i: (0, 0))


def _v_spec():
    return pl.BlockSpec((1, F), lambda i: (0, 0))


def _tc_encode(x, W1, b1, W2, b2, g, beta, B, rows=None, off=0):
    rows = x.shape[0] if rows is None else rows
    return pl.pallas_call(
        _enc_body,
        grid=(rows // B,),
        in_specs=[_row_spec(B, off), _w_spec(), _v_spec(), _w_spec(),
                  _v_spec(), _v_spec(), _v_spec()],
        out_specs=_row_spec(B),
        out_shape=jax.ShapeDtypeStruct((rows, F), jnp.float32),
    )(x, W1, b1.reshape(1, F), W2, b2.reshape(1, F),
      g.reshape(1, F), beta.reshape(1, F))


def _tc_edge(el, g1, g2, W1, b1, W2, b2, g, beta, last):
    body = _edge_body_last if last else _edge_body
    n_out = 1 if last else 2
    shp = jax.ShapeDtypeStruct((EH, F), jnp.float32)
    out = pl.pallas_call(
        body,
        grid=(EH // BE,),
        in_specs=[_row_spec(BE)] * 3 + [_w_spec(), _v_spec(), _w_spec(),
                                        _v_spec(), _v_spec(), _v_spec()],
        out_specs=[_row_spec(BE)] * n_out,
        out_shape=[shp] * n_out,
    )(el, g1, g2, W1, b1.reshape(1, F), W2, b2.reshape(1, F),
      g.reshape(1, F), beta.reshape(1, F))
    if last:
        return out[0], None
    return out[0], out[1]


def _tc_node(xl, pa, pb, W1, b1, W2, b2, g, beta):
    return pl.pallas_call(
        _node_body,
        grid=(N // BN,),
        in_specs=[_row_spec(BN)] * 5 + [_w_spec(), _v_spec(), _w_spec(),
                                        _v_spec(), _v_spec(), _v_spec()],
        out_specs=_row_spec(BN),
        out_shape=jax.ShapeDtypeStruct((N, F), jnp.float32),
    )(xl, pa[0], pa[1], pb[0], pb[1], W1, b1.reshape(1, F),
      W2, b2.reshape(1, F), g.reshape(1, F), beta.reshape(1, F))


def _tc_decode(xl, W1, b1, W2p, b2p):
    return pl.pallas_call(
        _dec_body,
        grid=(N // BN,),
        in_specs=[_row_spec(BN), _w_spec(), _v_spec(), _w_spec(), _v_spec()],
        out_specs=_row_spec(BN),
        out_shape=jax.ShapeDtypeStruct((N, F), jnp.float32),
    )(xl, W1, b1.reshape(1, F), W2p, b2p.reshape(1, F))


@functools.cache
def _sc_kernels():
    mesh = plsc.VectorSubcoreMesh(core_axis_name="c", subcore_axis_name="s",
                                  num_cores=NC, num_subcores=NS)

    @functools.partial(
        pl.kernel,
        out_type=jax.ShapeDtypeStruct((EH, F), jnp.float32),
        mesh=mesh,
        scratch_types=[
            pltpu.VMEM((NCH, CH), jnp.int32),
            pltpu.VMEM((2, ROWS, F), jnp.float32),
            pltpu.SemaphoreType.DMA,
            pltpu.SemaphoreType.DMA,
            pltpu.SemaphoreType.DMA,
            pltpu.SemaphoreType.DMA,
        ],
    )
    def sc_gather(xl_hbm, ei_hbm, g_hbm,
                  idx, buf, semg0, semg1, semw0, semw1):
        cid = lax.axis_index("c")
        sid = lax.axis_index("s")
        wid = sid * NC + cid
        base = wid * EPW
        pltpu.sync_copy(ei_hbm.at[wid], idx)
        semg = (semg0, semg1)
        semw = (semw0, semw1)

        def issue(grp, slot, sem):
            for b in range(GRP):
                pltpu.async_copy(xl_hbm.at[idx.at[grp * GRP + b]],
                                 buf.at[slot, pl.ds(b * CH, CH)], sem)

        def drain_gathers(slot, sem):
            for b in range(GRP):
                pltpu.make_async_copy(
                    xl_hbm.at[idx.at[0]],
                    buf.at[slot, pl.ds(b * CH, CH)], sem).wait()

        def drain_write(slot, sem):
            pltpu.make_async_copy(buf.at[slot],
                                  g_hbm.at[pl.ds(0, ROWS)], sem).wait()

        issue(0, 0, semg[0])

        @pl.loop(0, NOUT, step=2)
        def outer(j):
            for b in (0, 1):
                cur = j + b
                o = 1 - b

                @pl.when(cur < NOUT)
                def _():
                    @pl.when(cur >= 1)
                    def _():
                        drain_write(o, semw[o])

                    @pl.when(cur + 1 < NOUT)
                    def _():
                        issue(cur + 1, o, semg[o])

                    drain_gathers(b, semg[b])
                    pltpu.async_copy(buf.at[b],
                                     g_hbm.at[pl.ds(base + cur * ROWS, ROWS)],
                                     semw[b])

        drain_write((NOUT - 1) % 2, semw[(NOUT - 1) % 2])

    @functools.partial(
        pl.kernel,
        out_type=jax.ShapeDtypeStruct((NC, N, F), jnp.float32),
        mesh=mesh,
        scratch_types=[
            pltpu.VMEM((NCH, CH), jnp.int32),
            pltpu.VMEM((2, CH, F), jnp.float32),
            pltpu.VMEM_SHARED((N, F), jnp.float32),
            pltpu.SemaphoreType.DMA,
            pltpu.SemaphoreType.DMA,
            pltpu.SemaphoreType.DMA,
            pltpu.SemaphoreType.DMA,
        ],
    )
    def sc_scatter(enew_hbm, dst_hbm, zeros_hbm, out_hbm,
                   idx_d, buf, acc, semr0, semr1, sema0, sema1):
        cid = lax.axis_index("c")
        sid = lax.axis_index("s")
        wid = sid * NC + cid
        base = wid * EPW
        rps = N // 10
        @pl.when(sid < 10)
        def _zero():
            pltpu.sync_copy(zeros_hbm, acc.at[pl.ds(sid * rps, rps)])
        pltpu.sync_copy(dst_hbm.at[wid], idx_d)
        plsc.subcore_barrier()
        semr = (semr0, semr1)
        sema = (sema0, sema1)

        def issue(grp, slot):
            pltpu.async_copy(enew_hbm.at[pl.ds(base + grp * CH, CH)],
                             buf.at[slot], semr[slot])

        def drain_read(slot):
            pltpu.make_async_copy(enew_hbm.at[pl.ds(0, CH)],
                                  buf.at[slot], semr[slot]).wait()

        def drain_add(slot):
            pltpu.make_async_copy(buf.at[slot], acc.at[idx_d.at[0]],
                                  sema[slot]).wait()

        issue(0, 0)

        @pl.loop(0, NCH, step=2)
        def outer(j):
            for b in (0, 1):
                cur = j + b
                o = 1 - b

                @pl.when(cur < NCH)
                def _():
                    @pl.when(cur >= 1)
                    def _():
                        drain_add(o)

                    @pl.when(cur + 1 < NCH)
                    def _():
                        issue(cur + 1, o)

                    drain_read(b)
                    pltpu.async_copy(buf.at[b], acc.at[idx_d.at[cur]],
                                     sema[b], add=True)

        drain_add((NCH - 1) % 2)
        plsc.subcore_barrier()

        @pl.when(sid < 10)
        def _writeback():
            pltpu.sync_copy(acc.at[pl.ds(sid * rps, rps)],
                            out_hbm.at[cid, pl.ds(sid * rps, rps)])

    return sc_gather, sc_scatter


def kernel(x, edge_attr, en_W1, en_b1, en_W2, en_b2, en_g, en_beta,
           ee_W1, ee_b1, ee_W2, ee_b2, ee_g, ee_beta,
           pe_W1, pe_b1, pe_W2, pe_b2, pe_g, pe_beta,
           pn_W1, pn_b1, pn_W2, pn_b2, pn_g, pn_beta,
           d_W1, d_b1, d_W2, d_b2, edge_index):
    src = edge_index[0]
    dst = edge_index[1]
    src3 = [src[h * EH:(h + 1) * EH].reshape(NW, NCH, CH) for h in range(2)]
    dst3 = [dst[h * EH:(h + 1) * EH].reshape(NW, NCH, CH) for h in range(2)]
    zeros = jnp.zeros((N // 10, F), jnp.float32)

    xl = _tc_encode(x, en_W1, en_b1, en_W2, en_b2, en_g, en_beta, BN)
    el = [_tc_encode(edge_attr, ee_W1, ee_b1, ee_W2, ee_b2, ee_g, ee_beta,
                     BE, rows=EH, off=h * (EH // BE)) for h in range(2)]

    sc_gather, sc_scatter = _sc_kernels()
    for s in range(S):
        w = (pe_W1[s], pe_b1[s], pe_W2[s], pe_b2[s], pe_g[s], pe_beta[s])
        last = s == S - 1
        ga1 = sc_gather(xl, src3[0])
        ga2 = sc_gather(xl, dst3[0])
        gb1 = sc_gather(xl, src3[1])
        gb2 = sc_gather(xl, dst3[1])
        ea_new, ela = _tc_edge(el[0], ga1, ga2, *w, last=last)
        pa = sc_scatter(ea_new, dst3[0], zeros)
        eb_new, elb = _tc_edge(el[1], gb1, gb2, *w, last=last)
        pb = sc_scatter(eb_new, dst3[1], zeros)
        el = [ela, elb]
        xl = _tc_node(xl, pa, pb, pn_W1[s], pn_b1[s], pn_W2[s],
                      pn_b2[s], pn_g[s], pn_beta[s])

    W2p = jnp.pad(d_W2, ((0, 0), (0, F - OUT)))
    b2p = jnp.pad(d_b2, (0, F - OUT))
    out = _tc_decode(xl, d_W1, d_b1, W2p, b2p)
    return out[:, :OUT]


# speedup vs baseline: 3.0571x; 1.0010x over previous
import functools

import jax
import jax.numpy as jnp
from jax import lax
from jax.experimental import pallas as pl
from jax.experimental.pallas import tpu as pltpu
from jax.experimental.pallas import tpu_sc as plsc

N = 10000
E = 320000
F = 128
S = 5
OUT = 3

NC = 2
NS = 16
NW = NC * NS

EH = E // 2
EPW = EH // NW
CH = 40
NCH = EPW // CH
GRP = 5
ROWS = GRP * CH
NOUT = NCH // GRP

BE = 640
BN = 1000

_EPS = 1e-5


def _mlp_ln(xin, W1_ref, b1_ref, W2_ref, b2_ref, g_ref, beta_ref):
    h = jnp.maximum(
        jnp.dot(xin, W1_ref[...], preferred_element_type=jnp.float32)
        + b1_ref[...], 0.0)
    y = (jnp.dot(h, W2_ref[...], preferred_element_type=jnp.float32)
         + b2_ref[...])
    mu = jnp.mean(y, axis=-1, keepdims=True)
    d = y - mu
    var = jnp.mean(d * d, axis=-1, keepdims=True)
    return d * lax.rsqrt(var + _EPS) * g_ref[...] + beta_ref[...]


def _enc_body(x_ref, W1, b1, W2, b2, g, beta, o_ref):
    o_ref[...] = _mlp_ln(x_ref[...], W1, b1, W2, b2, g, beta)


def _edge_body(el_ref, g1_ref, g2_ref, W1, b1, W2, b2, g, beta,
               enew_ref, elnew_ref):
    el = el_ref[...]
    e_in = el + (g1_ref[...] + g2_ref[...])
    e_new = _mlp_ln(e_in, W1, b1, W2, b2, g, beta)
    enew_ref[...] = e_new
    elnew_ref[...] = el + e_new


def _edge_body_last(el_ref, g1_ref, g2_ref, W1, b1, W2, b2, g, beta,
                    enew_ref):
    e_in = el_ref[...] + (g1_ref[...] + g2_ref[...])
    enew_ref[...] = _mlp_ln(e_in, W1, b1, W2, b2, g, beta)


def _node_body(xl_ref, p0_ref, p1_ref, p2_ref, p3_ref,
               W1, b1, W2, b2, g, beta, o_ref):
    xl = xl_ref[...]
    t = xl + ((p0_ref[...] + p1_ref[...]) + (p2_ref[...] + p3_ref[...]))
    y = xl + _mlp_ln(t, W1, b1, W2, b2, g, beta)
    o_ref[...] = y


def _dec_body(xl_ref, W1, b1, W2, b2, o_ref):
    h = jnp.maximum(
        jnp.dot(xl_ref[...], W1[...], preferred_element_type=jnp.float32)
        + b1[...], 0.0)
    o_ref[...] = (jnp.dot(h, W2[...], preferred_element_type=jnp.float32)
                  + b2[...])


def _row_spec(B, off=0, w=F):
    return pl.BlockSpec((B, w), lambda i, _o=off: (i + _o, 0))


def _w_spec():
    return pl.BlockSpec((F, F), lambda i: (0, 0))


def _v_spec():
    return pl.BlockSpec((1, F), lambda i: (0, 0))


def _tc_encode(x, W1, b1, W2, b2, g, beta, B, rows=None, off=0):
    rows = x.shape[0] if rows is None else rows
    return pl.pallas_call(
        _enc_body,
        grid=(rows // B,),
        in_specs=[_row_spec(B, off), _w_spec(), _v_spec(), _w_spec(),
                  _v_spec(), _v_spec(), _v_spec()],
        out_specs=_row_spec(B),
        out_shape=jax.ShapeDtypeStruct((rows, F), jnp.float32),
    )(x, W1, b1.reshape(1, F), W2, b2.reshape(1, F),
      g.reshape(1, F), beta.reshape(1, F))


def _tc_edge(el, g1, g2, W1, b1, W2, b2, g, beta, last):
    body = _edge_body_last if last else _edge_body
    n_out = 1 if last else 2
    shp = jax.ShapeDtypeStruct((EH, F), jnp.float32)
    out = pl.pallas_call(
        body,
        grid=(EH // BE,),
        in_specs=[_row_spec(BE)] * 3 + [_w_spec(), _v_spec(), _w_spec(),
                                        _v_spec(), _v_spec(), _v_spec()],
        out_specs=[_row_spec(BE)] * n_out,
        out_shape=[shp] * n_out,
    )(el, g1, g2, W1, b1.reshape(1, F), W2, b2.reshape(1, F),
      g.reshape(1, F), beta.reshape(1, F))
    if last:
        return out[0], None
    return out[0], out[1]


def _tc_node(xl, pa, pb, W1, b1, W2, b2, g, beta):
    return pl.pallas_call(
        _node_body,
        grid=(N // BN,),
        in_specs=[_row_spec(BN)] * 5 + [_w_spec(), _v_spec(), _w_spec(),
                                        _v_spec(), _v_spec(), _v_spec()],
        out_specs=_row_spec(BN),
        out_shape=jax.ShapeDtypeStruct((N, F), jnp.float32),
    )(xl, pa[0], pa[1], pb[0], pb[1], W1, b1.reshape(1, F),
      W2, b2.reshape(1, F), g.reshape(1, F), beta.reshape(1, F))


def _tc_decode(xl, W1, b1, W2p, b2p):
    return pl.pallas_call(
        _dec_body,
        grid=(N // BN,),
        in_specs=[_row_spec(BN), _w_spec(), _v_spec(), _w_spec(), _v_spec()],
        out_specs=_row_spec(BN),
        out_shape=jax.ShapeDtypeStruct((N, F), jnp.float32),
    )(xl, W1, b1.reshape(1, F), W2p, b2p.reshape(1, F))


@functools.cache
def _sc_kernels():
    mesh = plsc.VectorSubcoreMesh(core_axis_name="c", subcore_axis_name="s",
                                  num_cores=NC, num_subcores=NS)

    NST = 2 * NOUT

    @functools.partial(
        pl.kernel,
        out_type=(jax.ShapeDtypeStruct((EH, F), jnp.float32),
                  jax.ShapeDtypeStruct((EH, F), jnp.float32)),
        mesh=mesh,
        scratch_types=[
            pltpu.VMEM((NCH, CH), jnp.int32),
            pltpu.VMEM((NCH, CH), jnp.int32),
            pltpu.VMEM((2, ROWS, F), jnp.float32),
            pltpu.SemaphoreType.DMA,
            pltpu.SemaphoreType.DMA,
            pltpu.SemaphoreType.DMA,
            pltpu.SemaphoreType.DMA,
        ],
    )
    def sc_gather(xl_hbm, src_hbm, dst_hbm, g1_hbm, g2_hbm,
                  idx_s, idx_d, buf, semg0, semg1, semw0, semw1):
        cid = lax.axis_index("c")
        sid = lax.axis_index("s")
        wid = sid * NC + cid
        base = wid * EPW
        pltpu.sync_copy(src_hbm.at[wid], idx_s)
        pltpu.sync_copy(dst_hbm.at[wid], idx_d)
        semg = (semg0, semg1)
        semw = (semw0, semw1)
        idx = (idx_s, idx_d)
        ghb = (g1_hbm, g2_hbm)

        def issue(stage, slot):
            for b in range(GRP):
                pltpu.async_copy(
                    xl_hbm.at[idx[slot].at[(stage // 2) * GRP + b]],
                    buf.at[slot, pl.ds(b * CH, CH)], semg[slot])

        def drain_gathers(slot):
            for b in range(GRP):
                pltpu.make_async_copy(
                    xl_hbm.at[idx_s.at[0]],
                    buf.at[slot, pl.ds(b * CH, CH)], semg[slot]).wait()

        def drain_write(slot):
            pltpu.make_async_copy(buf.at[slot],
                                  g1_hbm.at[pl.ds(0, ROWS)],
                                  semw[slot]).wait()

        issue(0, 0)

        @pl.loop(0, NST, step=2)
        def outer(j):
            for b in (0, 1):
                cur = j + b
                o = 1 - b

                @pl.when(cur < NST)
                def _():
                    @pl.when(cur >= 1)
                    def _():
                        drain_write(o)

                    @pl.when(cur + 1 < NST)
                    def _():
                        issue(cur + 1, o)

                    drain_gathers(b)
                    pltpu.async_copy(
                        buf.at[b],
                        ghb[b].at[pl.ds(base + (cur // 2) * ROWS, ROWS)],
                        semw[b])

        drain_write((NST - 1) % 2)

    @functools.partial(
        pl.kernel,
        out_type=jax.ShapeDtypeStruct((NC, N, F), jnp.float32),
        mesh=mesh,
        scratch_types=[
            pltpu.VMEM((NCH, CH), jnp.int32),
            pltpu.VMEM((2, CH, F), jnp.float32),
            pltpu.VMEM_SHARED((N, F), jnp.float32),
            pltpu.SemaphoreType.DMA,
            pltpu.SemaphoreType.DMA,
            pltpu.SemaphoreType.DMA,
            pltpu.SemaphoreType.DMA,
        ],
    )
    def sc_scatter(enew_hbm, dst_hbm, zeros_hbm, out_hbm,
                   idx_d, buf, acc, semr0, semr1, sema0, sema1):
        cid = lax.axis_index("c")
        sid = lax.axis_index("s")
        wid = sid * NC + cid
        base = wid * EPW
        rps = N // 10
        @pl.when(sid < 10)
        def _zero():
            pltpu.sync_copy(zeros_hbm, acc.at[pl.ds(sid * rps, rps)])
        pltpu.sync_copy(dst_hbm.at[wid], idx_d)
        plsc.subcore_barrier()
        semr = (semr0, semr1)
        sema = (sema0, sema1)

        def issue(grp, slot):
            pltpu.async_copy(enew_hbm.at[pl.ds(base + grp * CH, CH)],
                             buf.at[slot], semr[slot])

        def drain_read(slot):
            pltpu.make_async_copy(enew_hbm.at[pl.ds(0, CH)],
                                  buf.at[slot], semr[slot]).wait()

        def drain_add(slot):
            pltpu.make_async_copy(buf.at[slot], acc.at[idx_d.at[0]],
                                  sema[slot]).wait()

        issue(0, 0)

        @pl.loop(0, NCH, step=2)
        def outer(j):
            for b in (0, 1):
                cur = j + b
                o = 1 - b

                @pl.when(cur < NCH)
                def _():
                    @pl.when(cur >= 1)
                    def _():
                        drain_add(o)

                    @pl.when(cur + 1 < NCH)
                    def _():
                        issue(cur + 1, o)

                    drain_read(b)
                    pltpu.async_copy(buf.at[b], acc.at[idx_d.at[cur]],
                                     sema[b], add=True)

        drain_add((NCH - 1) % 2)
        plsc.subcore_barrier()

        @pl.when(sid < 10)
        def _writeback():
            pltpu.sync_copy(acc.at[pl.ds(sid * rps, rps)],
                            out_hbm.at[cid, pl.ds(sid * rps, rps)])

    return sc_gather, sc_scatter


def kernel(x, edge_attr, en_W1, en_b1, en_W2, en_b2, en_g, en_beta,
           ee_W1, ee_b1, ee_W2, ee_b2, ee_g, ee_beta,
           pe_W1, pe_b1, pe_W2, pe_b2, pe_g, pe_beta,
           pn_W1, pn_b1, pn_W2, pn_b2, pn_g, pn_beta,
           d_W1, d_b1, d_W2, d_b2, edge_index):
    src = edge_index[0]
    dst = edge_index[1]
    src3 = [src[h * EH:(h + 1) * EH].reshape(NW, NCH, CH) for h in range(2)]
    dst3 = [dst[h * EH:(h + 1) * EH].reshape(NW, NCH, CH) for h in range(2)]
    zeros = jnp.zeros((N // 10, F), jnp.float32)

    xl = _tc_encode(x, en_W1, en_b1, en_W2, en_b2, en_g, en_beta, BN)
    el = [_tc_encode(edge_attr, ee_W1, ee_b1, ee_W2, ee_b2, ee_g, ee_beta,
                     BE, rows=EH, off=h * (EH // BE)) for h in range(2)]

    sc_gather, sc_scatter = _sc_kernels()
    for s in range(S):
        w = (pe_W1[s], pe_b1[s], pe_W2[s], pe_b2[s], pe_g[s], pe_beta[s])
        last = s == S - 1
        ga1, ga2 = sc_gather(xl, src3[0], dst3[0])
        gb1, gb2 = sc_gather(xl, src3[1], dst3[1])
        ea_new, ela = _tc_edge(el[0], ga1, ga2, *w, last=last)
        pa = sc_scatter(ea_new, dst3[0], zeros)
        eb_new, elb = _tc_edge(el[1], gb1, gb2, *w, last=last)
        pb = sc_scatter(eb_new, dst3[1], zeros)
        el = [ela, elb]
        xl = _tc_node(xl, pa, pb, pn_W1[s], pn_b1[s], pn_W2[s],
                      pn_b2[s], pn_g[s], pn_beta[s])

    W2p = jnp.pad(d_W2, ((0, 0), (0, F - OUT)))
    b2p = jnp.pad(d_b2, (0, F - OUT))
    out = _tc_decode(xl, d_W1, d_b1, W2p, b2p)
    return out[:, :OUT]


# speedup vs baseline: 3.0575x; 1.0001x over previous
import functools

import jax
import jax.numpy as jnp
from jax import lax
from jax.experimental import pallas as pl
from jax.experimental.pallas import tpu as pltpu
from jax.experimental.pallas import tpu_sc as plsc

N = 10000
E = 320000
F = 128
S = 5
OUT = 3

NC = 2
NS = 16
NW = NC * NS

EH = E // 2
EPW = EH // NW
CH = 40
NCH = EPW // CH
GRP = 5
ROWS = GRP * CH
NOUT = NCH // GRP

BE = 640
BN = 1000

_EPS = 1e-5


def _mlp_ln(xin, W1_ref, b1_ref, W2_ref, b2_ref, g_ref, beta_ref):
    h = jnp.maximum(
        jnp.dot(xin, W1_ref[...], preferred_element_type=jnp.float32)
        + b1_ref[...], 0.0)
    y = (jnp.dot(h, W2_ref[...], preferred_element_type=jnp.float32)
         + b2_ref[...])
    mu = jnp.mean(y, axis=-1, keepdims=True)
    d = y - mu
    var = jnp.mean(d * d, axis=-1, keepdims=True)
    return d * lax.rsqrt(var + _EPS) * g_ref[...] + beta_ref[...]


def _enc_body(x_ref, W1, b1, W2, b2, g, beta, o_ref):
    o_ref[...] = _mlp_ln(x_ref[...], W1, b1, W2, b2, g, beta)


def _edge_body(el_ref, g1_ref, g2_ref, W1, b1, W2, b2, g, beta,
               enew_ref, elnew_ref):
    el = el_ref[...]
    e_in = el + (g1_ref[...] + g2_ref[...])
    e_new = _mlp_ln(e_in, W1, b1, W2, b2, g, beta)
    enew_ref[...] = e_new
    elnew_ref[...] = el + e_new


def _edge_body_last(el_ref, g1_ref, g2_ref, W1, b1, W2, b2, g, beta,
                    enew_ref):
    e_in = el_ref[...] + (g1_ref[...] + g2_ref[...])
    enew_ref[...] = _mlp_ln(e_in, W1, b1, W2, b2, g, beta)


def _node_body(xl_ref, p0_ref, p1_ref, p2_ref, p3_ref,
               W1, b1, W2, b2, g, beta, o_ref):
    xl = xl_ref[...]
    t = xl + ((p0_ref[...] + p1_ref[...]) + (p2_ref[...] + p3_ref[...]))
    y = xl + _mlp_ln(t, W1, b1, W2, b2, g, beta)
    o_ref[...] = y


def _dec_body(xl_ref, W1, b1, W2, b2, o_ref):
    h = jnp.maximum(
        jnp.dot(xl_ref[...], W1[...], preferred_element_type=jnp.float32)
        + b1[...], 0.0)
    o_ref[...] = (jnp.dot(h, W2[...], preferred_element_type=jnp.float32)
                  + b2[...])


def _row_spec(B, off=0, w=F):
    return pl.BlockSpec((B, w), lambda i, _o=off: (i + _o, 0))


def _w_spec():
    return pl.BlockSpec((F, F), lambda i: (0, 0))


def _v_spec():
    return pl.BlockSpec((1, F), lambda i: (0, 0))


def _tc_encode(x, W1, b1, W2, b2, g, beta, B, rows=None, off=0):
    rows = x.shape[0] if rows is None else rows
    return pl.pallas_call(
        _enc_body,
        grid=(rows // B,),
        in_specs=[_row_spec(B, off), _w_spec(), _v_spec(), _w_spec(),
                  _v_spec(), _v_spec(), _v_spec()],
        out_specs=_row_spec(B),
        out_shape=jax.ShapeDtypeStruct((rows, F), jnp.float32),
    )(x, W1, b1.reshape(1, F), W2, b2.reshape(1, F),
      g.reshape(1, F), beta.reshape(1, F))


def _tc_edge(el, g1, g2, W1, b1, W2, b2, g, beta, last):
    body = _edge_body_last if last else _edge_body
    n_out = 1 if last else 2
    shp = jax.ShapeDtypeStruct((EH, F), jnp.float32)
    out = pl.pallas_call(
        body,
        grid=(EH // BE,),
        in_specs=[_row_spec(BE)] * 3 + [_w_spec(), _v_spec(), _w_spec(),
                                        _v_spec(), _v_spec(), _v_spec()],
        out_specs=[_row_spec(BE)] * n_out,
        out_shape=[shp] * n_out,
    )(el, g1, g2, W1, b1.reshape(1, F), W2, b2.reshape(1, F),
      g.reshape(1, F), beta.reshape(1, F))
    if last:
        return out[0], None
    return out[0], out[1]


def _tc_node(xl, pa, pb, W1, b1, W2, b2, g, beta):
    return pl.pallas_call(
        _node_body,
        grid=(N // BN,),
        in_specs=[_row_spec(BN)] * 5 + [_w_spec(), _v_spec(), _w_spec(),
                                        _v_spec(), _v_spec(), _v_spec()],
        out_specs=_row_spec(BN),
        out_shape=jax.ShapeDtypeStruct((N, F), jnp.float32),
    )(xl, pa[0], pa[1], pb[0], pb[1], W1, b1.reshape(1, F),
      W2, b2.reshape(1, F), g.reshape(1, F), beta.reshape(1, F))


def _tc_decode(xl, W1, b1, W2p, b2p):
    return pl.pallas_call(
        _dec_body,
        grid=(N // BN,),
        in_specs=[_row_spec(BN), _w_spec(), _v_spec(), _w_spec(), _v_spec()],
        out_specs=_row_spec(BN),
        out_shape=jax.ShapeDtypeStruct((N, F), jnp.float32),
    )(xl, W1, b1.reshape(1, F), W2p, b2p.reshape(1, F))


@functools.cache
def _sc_kernels():
    mesh = plsc.VectorSubcoreMesh(core_axis_name="c", subcore_axis_name="s",
                                  num_cores=NC, num_subcores=NS)

    NST = 2 * NOUT

    @functools.partial(
        pl.kernel,
        out_type=(jax.ShapeDtypeStruct((EH, F), jnp.float32),
                  jax.ShapeDtypeStruct((EH, F), jnp.float32)),
        mesh=mesh,
        scratch_types=[
            pltpu.VMEM((NCH, CH), jnp.int32),
            pltpu.VMEM((NCH, CH), jnp.int32),
            pltpu.VMEM((3, ROWS, F), jnp.float32),
            pltpu.SemaphoreType.DMA,
            pltpu.SemaphoreType.DMA,
            pltpu.SemaphoreType.DMA,
            pltpu.SemaphoreType.DMA,
            pltpu.SemaphoreType.DMA,
            pltpu.SemaphoreType.DMA,
        ],
    )
    def sc_gather(xl_hbm, src_hbm, dst_hbm, g1_hbm, g2_hbm,
                  idx_s, idx_d, buf, semg0, semg1, semg2,
                  semw0, semw1, semw2):
        cid = lax.axis_index("c")
        sid = lax.axis_index("s")
        wid = sid * NC + cid
        base = wid * EPW
        pltpu.sync_copy(src_hbm.at[wid], idx_s)
        pltpu.sync_copy(dst_hbm.at[wid], idx_d)
        semg = (semg0, semg1, semg2)
        semw = (semw0, semw1, semw2)
        idx = (idx_s, idx_d)
        ghb = (g1_hbm, g2_hbm)

        def issue(grp_i, stream, slot):
            for b in range(GRP):
                pltpu.async_copy(
                    xl_hbm.at[idx[stream].at[grp_i * GRP + b]],
                    buf.at[slot, pl.ds(b * CH, CH)], semg[slot])

        def drain_gathers(slot):
            for b in range(GRP):
                pltpu.make_async_copy(
                    xl_hbm.at[idx_s.at[0]],
                    buf.at[slot, pl.ds(b * CH, CH)], semg[slot]).wait()

        def drain_write(slot):
            pltpu.make_async_copy(buf.at[slot],
                                  g1_hbm.at[pl.ds(0, ROWS)],
                                  semw[slot]).wait()

        issue(0, 0, 0)
        issue(0, 1, 1)

        @pl.loop(0, NST, step=6)
        def outer(j):
            for b in range(6):
                cur = j + b
                slot = b % 3
                nslot = (b + 2) % 3

                @pl.when(cur < NST)
                def _():
                    @pl.when(cur >= 1)
                    def _():
                        drain_write(nslot)

                    @pl.when(cur + 2 < NST)
                    def _():
                        issue((cur + 2) // 2, b % 2, nslot)

                    drain_gathers(slot)
                    pltpu.async_copy(
                        buf.at[slot],
                        ghb[b % 2].at[pl.ds(base + (cur // 2) * ROWS,
                                            ROWS)],
                        semw[slot])

        drain_write((NST - 1) % 3)

    @functools.partial(
        pl.kernel,
        out_type=jax.ShapeDtypeStruct((NC, N, F), jnp.float32),
        mesh=mesh,
        scratch_types=[
            pltpu.VMEM((NCH, CH), jnp.int32),
            pltpu.VMEM((2, CH, F), jnp.float32),
            pltpu.VMEM_SHARED((N, F), jnp.float32),
            pltpu.SemaphoreType.DMA,
            pltpu.SemaphoreType.DMA,
            pltpu.SemaphoreType.DMA,
            pltpu.SemaphoreType.DMA,
        ],
    )
    def sc_scatter(enew_hbm, dst_hbm, zeros_hbm, out_hbm,
                   idx_d, buf, acc, semr0, semr1, sema0, sema1):
        cid = lax.axis_index("c")
        sid = lax.axis_index("s")
        wid = sid * NC + cid
        base = wid * EPW
        rps = N // 10
        @pl.when(sid < 10)
        def _zero():
            pltpu.sync_copy(zeros_hbm, acc.at[pl.ds(sid * rps, rps)])
        pltpu.sync_copy(dst_hbm.at[wid], idx_d)
        plsc.subcore_barrier()
        semr = (semr0, semr1)
        sema = (sema0, sema1)

        def issue(grp, slot):
            pltpu.async_copy(enew_hbm.at[pl.ds(base + grp * CH, CH)],
                             buf.at[slot], semr[slot])

        def drain_read(slot):
            pltpu.make_async_copy(enew_hbm.at[pl.ds(0, CH)],
                                  buf.at[slot], semr[slot]).wait()

        def drain_add(slot):
            pltpu.make_async_copy(buf.at[slot], acc.at[idx_d.at[0]],
                                  sema[slot]).wait()

        issue(0, 0)

        @pl.loop(0, NCH, step=2)
        def outer(j):
            for b in (0, 1):
                cur = j + b
                o = 1 - b

                @pl.when(cur < NCH)
                def _():
                    @pl.when(cur >= 1)
                    def _():
                        drain_add(o)

                    @pl.when(cur + 1 < NCH)
                    def _():
                        issue(cur + 1, o)

                    drain_read(b)
                    pltpu.async_copy(buf.at[b], acc.at[idx_d.at[cur]],
                                     sema[b], add=True)

        drain_add((NCH - 1) % 2)
        plsc.subcore_barrier()

        @pl.when(sid < 10)
        def _writeback():
            pltpu.sync_copy(acc.at[pl.ds(sid * rps, rps)],
                            out_hbm.at[cid, pl.ds(sid * rps, rps)])

    return sc_gather, sc_scatter


def kernel(x, edge_attr, en_W1, en_b1, en_W2, en_b2, en_g, en_beta,
           ee_W1, ee_b1, ee_W2, ee_b2, ee_g, ee_beta,
           pe_W1, pe_b1, pe_W2, pe_b2, pe_g, pe_beta,
           pn_W1, pn_b1, pn_W2, pn_b2, pn_g, pn_beta,
           d_W1, d_b1, d_W2, d_b2, edge_index):
    src = edge_index[0]
    dst = edge_index[1]
    src3 = [src[h * EH:(h + 1) * EH].reshape(NW, NCH, CH) for h in range(2)]
    dst3 = [dst[h * EH:(h + 1) * EH].reshape(NW, NCH, CH) for h in range(2)]
    zeros = jnp.zeros((N // 10, F), jnp.float32)

    xl = _tc_encode(x, en_W1, en_b1, en_W2, en_b2, en_g, en_beta, BN)
    el = [_tc_encode(edge_attr, ee_W1, ee_b1, ee_W2, ee_b2, ee_g, ee_beta,
                     BE, rows=EH, off=h * (EH // BE)) for h in range(2)]

    sc_gather, sc_scatter = _sc_kernels()
    for s in range(S):
        w = (pe_W1[s], pe_b1[s], pe_W2[s], pe_b2[s], pe_g[s], pe_beta[s])
        last = s == S - 1
        ga1, ga2 = sc_gather(xl, src3[0], dst3[0])
        gb1, gb2 = sc_gather(xl, src3[1], dst3[1])
        ea_new, ela = _tc_edge(el[0], ga1, ga2, *w, last=last)
        pa = sc_scatter(ea_new, dst3[0], zeros)
        eb_new, elb = _tc_edge(el[1], gb1, gb2, *w, last=last)
        pb = sc_scatter(eb_new, dst3[1], zeros)
        el = [ela, elb]
        xl = _tc_node(xl, pa, pb, pn_W1[s], pn_b1[s], pn_W2[s],
                      pn_b2[s], pn_g[s], pn_beta[s])

    W2p = jnp.pad(d_W2, ((0, 0), (0, F - OUT)))
    b2p = jnp.pad(d_b2, (0, F - OUT))
    out = _tc_decode(xl, d_W1, d_b1, W2p, b2p)
    return out[:, :OUT]
